# baseline (speedup 1.0000x reference)
import functools

import jax
import jax.numpy as jnp
from jax import lax
from jax.experimental import pallas as pl
from jax.experimental.pallas import tpu as pltpu
from jax.experimental.pallas import tpu_sc as plsc

END_ID = 2
B = 128
V = 100000
T = 2048
NC = 2
NS = 16
L = 16
NW = NC * NS
BG = B // L

VT = 49152
CV = 6144
NB = VT // CV
VTAIL = V - VT
SPAN = 1592
V0_LAST = VTAIL - SPAN
CHUNKS = (400, 400, 400, 392)
CH_MAX = max(CHUNKS)

_mesh = plsc.VectorSubcoreMesh(core_axis_name="c", subcore_axis_name="s")


@functools.partial(
    pl.kernel,
    out_type=[
        jax.ShapeDtypeStruct((NW, B), jnp.float32),
        jax.ShapeDtypeStruct((NW, B), jnp.int32),
    ],
    mesh=_mesh,
    compiler_params=pltpu.CompilerParams(
        needs_layout_passes=False, skip_device_barrier=True),
    scratch_types=[
        pltpu.VMEM((CH_MAX, B), jnp.float32),
        pltpu.VMEM((CH_MAX, B), jnp.float32),
        pltpu.VMEM((B,), jnp.float32),
        pltpu.VMEM((B,), jnp.int32),
        pltpu.SemaphoreType.DMA,
        pltpu.SemaphoreType.DMA,
    ],
)
def _sc_scan(hid, valstage, idxstage, buf0, buf1, vrow, irow, sem0, sem1):
    wid = lax.axis_index("s") * NC + lax.axis_index("c")
    bufs, sems = (buf0, buf1), (sem0, sem1)
    v0 = pl.multiple_of(VT + jnp.minimum(wid * SPAN, V0_LAST), 8)

    offs = [0]
    for c in CHUNKS:
        offs.append(offs[-1] + c)

    def start(k):
        n = CHUNKS[k]
        return pltpu.async_copy(
            hid.at[pl.ds(v0 + offs[k], n)],
            bufs[k % 2].at[pl.ds(0, n)], sems[k % 2])

    vmaxs = [jnp.full((L,), -jnp.inf, jnp.float32) for _ in range(BG)]
    vjs = [jnp.zeros((L,), jnp.int32) for _ in range(BG)]

    handle = start(0)
    for k, n in enumerate(CHUNKS):
        nxt = start(k + 1) if k + 1 < len(CHUNKS) else None
        handle.wait()
        handle = nxt
        buf = bufs[k % 2]
        vbase = v0 + offs[k]

        def body(i, carry):
            vm, vj = carry
            jvec = jnp.full((L,), vbase + i, jnp.int32)
            nvm, nvj = [], []
            for g in range(BG):
                v = buf[i, pl.ds(g * L, L)]
                msk = v > vm[g]
                nvm.append(jnp.where(msk, v, vm[g]))
                nvj.append(jnp.where(msk, jvec, vj[g]))
            return tuple(nvm), tuple(nvj)

        vmaxs, vjs = plsc.parallel_loop(
            0, n, 1, unroll=2, carry=(tuple(vmaxs), tuple(vjs)))(body)
        vmaxs, vjs = list(vmaxs), list(vjs)

    for g in range(BG):
        vrow[pl.ds(g * L, L)] = vmaxs[g]
        irow[pl.ds(g * L, L)] = vjs[g]
    pltpu.sync_copy(vrow, valstage.at[wid])
    pltpu.sync_copy(irow, idxstage.at[wid])


def _tc_head_body(hid_ref, mout, iout):
    b = pl.program_id(0)
    x = hid_ref[...]
    bm = jnp.max(x, axis=0, keepdims=True)
    iota = lax.broadcasted_iota(jnp.int32, (CV, B), 0)
    bi = jnp.min(jnp.where(x == bm, iota, jnp.int32(2**31 - 1)),
                 axis=0, keepdims=True)
    mout[...] = bm.reshape(1, 1, B)
    iout[...] = (bi + b * CV).reshape(1, 1, B)


_tc_head = pl.pallas_call(
    _tc_head_body,
    grid=(NB,),
    in_specs=[pl.BlockSpec((CV, B), lambda i: (i, 0))],
    out_specs=[pl.BlockSpec((1, 1, B), lambda i: (i, 0, 0)),
               pl.BlockSpec((1, 1, B), lambda i: (i, 0, 0))],
    out_shape=[jax.ShapeDtypeStruct((NB, 1, B), jnp.float32),
               jax.ShapeDtypeStruct((NB, 1, B), jnp.int32)],
    compiler_params=pltpu.CompilerParams(
        dimension_semantics=("arbitrary",)),
)


def _tc_merge_body(scm, sci, tcm, tci, flg, upd, outin, out_ref, flout):
    m = tcm[0]
    g = tci[0]
    for bb in range(1, NB):
        m2 = tcm[bb]
        g2 = tci[bb]
        better = (m2 > m) | ((m2 == m) & (g2 < g))
        m = jnp.where(better, m2, m)
        g = jnp.where(better, g2, g)
    for w in range(NW):
        m2 = scm[pl.ds(w, 1), :]
        g2 = sci[pl.ds(w, 1), :]
        better = (m2 > m) | ((m2 == m) & (g2 < g))
        m = jnp.where(better, m2, m)
        g = jnp.where(better, g2, g)
    y = jnp.where(flg[...] != 0, g, jnp.full((1, B), END_ID, jnp.int32))
    flout[...] = (y != END_ID).astype(jnp.int32)
    out_ref[...] = outin[...]
    u = upd[0]
    base = pl.multiple_of((u // 128) * 128, 128)
    win = out_ref[:, pl.ds(base, 128)]
    cols = lax.broadcasted_iota(jnp.int32, (B, 128), 1)
    ybc = jnp.broadcast_to(y.reshape(B, 1), (B, 128))
    out_ref[:, pl.ds(base, 128)] = jnp.where(cols == (u - base), ybc, win)


_tc_merge = pl.pallas_call(
    _tc_merge_body,
    in_specs=[pl.BlockSpec((NW, B), lambda: (0, 0)),
              pl.BlockSpec((NW, B), lambda: (0, 0)),
              pl.BlockSpec((NB, 1, B), lambda: (0, 0, 0)),
              pl.BlockSpec((NB, 1, B), lambda: (0, 0, 0)),
              pl.BlockSpec((1, B), lambda: (0, 0)),
              pl.BlockSpec(memory_space=pltpu.SMEM),
              pl.BlockSpec((B, T), lambda: (0, 0))],
    out_specs=[pl.BlockSpec((B, T), lambda: (0, 0)),
               pl.BlockSpec((1, B), lambda: (0, 0))],
    out_shape=[jax.ShapeDtypeStruct((B, T), jnp.int32),
               jax.ShapeDtypeStruct((1, B), jnp.int32)],
)


def kernel(hidden_state, update_index, out_ids, flags):
    hid = jnp.transpose(hidden_state, (1, 2, 0)).reshape(V, B)
    upd = jnp.asarray(update_index, jnp.int32).reshape(1)
    flags128 = flags.reshape(1, B).astype(jnp.int32)
    scm, sci = _sc_scan(hid)
    tcm, tci = _tc_head(hid)
    out, flout = _tc_merge(scm, sci, tcm, tci, flags128, upd, out_ids)
    flags_new = flout.reshape(B, 1).astype(jnp.bool_)
    return out, flags_new


# speedup vs baseline: 1.0036x; 1.0036x over previous
import functools

import jax
import jax.numpy as jnp
from jax import lax
from jax.experimental import pallas as pl
from jax.experimental.pallas import tpu as pltpu
from jax.experimental.pallas import tpu_sc as plsc

END_ID = 2
B = 128
V = 100000
T = 2048
NC = 2
NS = 16
L = 16
NW = NC * NS
BG = B // L

VT = 55296
CV = 6912
NB = VT // CV
VTAIL = V - VT
SPAN = 1400
V0_LAST = VTAIL - SPAN
CHUNKS = (360, 360, 360, 320)
CH_MAX = max(CHUNKS)

_mesh = plsc.VectorSubcoreMesh(core_axis_name="c", subcore_axis_name="s")


@functools.partial(
    pl.kernel,
    out_type=[
        jax.ShapeDtypeStruct((NW, B), jnp.float32),
        jax.ShapeDtypeStruct((NW, B), jnp.int32),
    ],
    mesh=_mesh,
    compiler_params=pltpu.CompilerParams(
        needs_layout_passes=False, skip_device_barrier=True),
    scratch_types=[
        pltpu.VMEM((CH_MAX, B), jnp.float32),
        pltpu.VMEM((CH_MAX, B), jnp.float32),
        pltpu.VMEM((B,), jnp.float32),
        pltpu.VMEM((B,), jnp.int32),
        pltpu.SemaphoreType.DMA,
        pltpu.SemaphoreType.DMA,
    ],
)
def _sc_scan(hid, valstage, idxstage, buf0, buf1, vrow, irow, sem0, sem1):
    wid = lax.axis_index("s") * NC + lax.axis_index("c")
    bufs, sems = (buf0, buf1), (sem0, sem1)
    v0 = pl.multiple_of(VT + jnp.minimum(wid * SPAN, V0_LAST), 8)

    offs = [0]
    for c in CHUNKS:
        offs.append(offs[-1] + c)

    def start(k):
        n = CHUNKS[k]
        return pltpu.async_copy(
            hid.at[pl.ds(v0 + offs[k], n)],
            bufs[k % 2].at[pl.ds(0, n)], sems[k % 2])

    vmaxs = [jnp.full((L,), -jnp.inf, jnp.float32) for _ in range(BG)]
    vjs = [jnp.zeros((L,), jnp.int32) for _ in range(BG)]

    handle = start(0)
    for k, n in enumerate(CHUNKS):
        nxt = start(k + 1) if k + 1 < len(CHUNKS) else None
        handle.wait()
        handle = nxt
        buf = bufs[k % 2]
        vbase = v0 + offs[k]

        def body(i, carry):
            vm, vj = carry
            jvec = jnp.full((L,), vbase + i, jnp.int32)
            nvm, nvj = [], []
            for g in range(BG):
                v = buf[i, pl.ds(g * L, L)]
                msk = v > vm[g]
                nvm.append(jnp.where(msk, v, vm[g]))
                nvj.append(jnp.where(msk, jvec, vj[g]))
            return tuple(nvm), tuple(nvj)

        vmaxs, vjs = plsc.parallel_loop(
            0, n, 1, unroll=2, carry=(tuple(vmaxs), tuple(vjs)))(body)
        vmaxs, vjs = list(vmaxs), list(vjs)

    for g in range(BG):
        vrow[pl.ds(g * L, L)] = vmaxs[g]
        irow[pl.ds(g * L, L)] = vjs[g]
    pltpu.sync_copy(vrow, valstage.at[wid])
    pltpu.sync_copy(irow, idxstage.at[wid])


def _tc_head_body(hid_ref, mout, iout):
    b = pl.program_id(0)
    x = hid_ref[...]
    bm = jnp.max(x, axis=0, keepdims=True)
    iota = lax.broadcasted_iota(jnp.int32, (CV, B), 0)
    bi = jnp.min(jnp.where(x == bm, iota, jnp.int32(2**31 - 1)),
                 axis=0, keepdims=True)
    mout[...] = bm.reshape(1, 1, B)
    iout[...] = (bi + b * CV).reshape(1, 1, B)


_tc_head = pl.pallas_call(
    _tc_head_body,
    grid=(NB,),
    in_specs=[pl.BlockSpec((CV, B), lambda i: (i, 0))],
    out_specs=[pl.BlockSpec((1, 1, B), lambda i: (i, 0, 0)),
               pl.BlockSpec((1, 1, B), lambda i: (i, 0, 0))],
    out_shape=[jax.ShapeDtypeStruct((NB, 1, B), jnp.float32),
               jax.ShapeDtypeStruct((NB, 1, B), jnp.int32)],
    compiler_params=pltpu.CompilerParams(
        dimension_semantics=("arbitrary",)),
)


def _tc_merge_body(scm, sci, tcm, tci, flg, upd, outin, out_ref, flout):
    m = tcm[0]
    g = tci[0]
    for bb in range(1, NB):
        m2 = tcm[bb]
        g2 = tci[bb]
        better = (m2 > m) | ((m2 == m) & (g2 < g))
        m = jnp.where(better, m2, m)
        g = jnp.where(better, g2, g)
    for w in range(NW):
        m2 = scm[pl.ds(w, 1), :]
        g2 = sci[pl.ds(w, 1), :]
        better = (m2 > m) | ((m2 == m) & (g2 < g))
        m = jnp.where(better, m2, m)
        g = jnp.where(better, g2, g)
    y = jnp.where(flg[...] != 0, g, jnp.full((1, B), END_ID, jnp.int32))
    flout[...] = (y != END_ID).astype(jnp.int32)
    out_ref[...] = outin[...]
    u = upd[0]
    base = pl.multiple_of((u // 128) * 128, 128)
    win = out_ref[:, pl.ds(base, 128)]
    cols = lax.broadcasted_iota(jnp.int32, (B, 128), 1)
    ybc = jnp.broadcast_to(y.reshape(B, 1), (B, 128))
    out_ref[:, pl.ds(base, 128)] = jnp.where(cols == (u - base), ybc, win)


_tc_merge = pl.pallas_call(
    _tc_merge_body,
    in_specs=[pl.BlockSpec((NW, B), lambda: (0, 0)),
              pl.BlockSpec((NW, B), lambda: (0, 0)),
              pl.BlockSpec((NB, 1, B), lambda: (0, 0, 0)),
              pl.BlockSpec((NB, 1, B), lambda: (0, 0, 0)),
              pl.BlockSpec((1, B), lambda: (0, 0)),
              pl.BlockSpec(memory_space=pltpu.SMEM),
              pl.BlockSpec((B, T), lambda: (0, 0))],
    out_specs=[pl.BlockSpec((B, T), lambda: (0, 0)),
               pl.BlockSpec((1, B), lambda: (0, 0))],
    out_shape=[jax.ShapeDtypeStruct((B, T), jnp.int32),
               jax.ShapeDtypeStruct((1, B), jnp.int32)],
)


def kernel(hidden_state, update_index, out_ids, flags):
    hid = jnp.transpose(hidden_state, (1, 2, 0)).reshape(V, B)
    upd = jnp.asarray(update_index, jnp.int32).reshape(1)
    flags128 = flags.reshape(1, B).astype(jnp.int32)
    scm, sci = _sc_scan(hid)
    tcm, tci = _tc_head(hid)
    out, flout = _tc_merge(scm, sci, tcm, tci, flags128, upd, out_ids)
    flags_new = flout.reshape(B, 1).astype(jnp.bool_)
    return out, flags_new
